# baseline (speedup 1.0000x reference)
import jax
import jax.numpy as jnp
from jax import lax
from jax.experimental import pallas as pl
from jax.experimental.pallas import tpu as pltpu
from jax.experimental.pallas import tpu_sc as plsc

VOCAB = 100000
HIDDEN = 128
BATCH = 16384
HIST = 50

_INFO = plsc.get_sparse_core_info()
_NC = _INFO.num_cores
_NS = _INFO.num_subcores
_NW = _NC * _NS

_B = BATCH * HIST
_PER_W = _B // _NW
_NBUF = 8
_CHUNK = 80
_NCHUNKS = _PER_W // _CHUNK
_NGROUPS = _NCHUNKS // _NBUF


def _gather_body(idx_hbm, table_hbm, out_hbm, idx_v, *bufs_and_sems):
    bufs = bufs_and_sems[:_NBUF]
    gsems = bufs_and_sems[_NBUF:2 * _NBUF]
    ssems = bufs_and_sems[2 * _NBUF:3 * _NBUF]

    wid = lax.axis_index("s") * _NC + lax.axis_index("c")
    base = wid * _PER_W

    pltpu.sync_copy(idx_hbm.at[pl.ds(base, _PER_W)], idx_v)

    def start_gather(i, b):
        pltpu.async_copy(table_hbm.at[idx_v.at[pl.ds(i * _CHUNK, _CHUNK)]],
                         bufs[b], gsems[b])

    def wait_gather(b):
        pltpu.make_async_copy(table_hbm.at[idx_v.at[pl.ds(0, _CHUNK)]],
                              bufs[b], gsems[b]).wait()

    def start_store(i, b):
        pltpu.async_copy(bufs[b], out_hbm.at[pl.ds(base + i * _CHUNK, _CHUNK)],
                         ssems[b])

    def wait_store(b):
        pltpu.make_async_copy(bufs[b], out_hbm.at[pl.ds(base, _CHUNK)],
                              ssems[b]).wait()

    for b in range(_NBUF - 1):
        start_gather(b, b)

    def group(g, carry):
        for b in range(_NBUF):
            i = g * _NBUF + b
            wait_gather(b)
            start_store(i, b)
            j = i + _NBUF - 1
            jb = (b + _NBUF - 1) % _NBUF

            def issue_next(j=j, jb=jb, b=b):
                if b == 0:
                    @pl.when(g > 0)
                    def _():
                        wait_store(jb)
                else:
                    wait_store(jb)
                start_gather(j, jb)

            @pl.when(j < _NCHUNKS)
            def _():
                issue_next()
        return carry

    lax.fori_loop(0, _NGROUPS, group, 0)
    for b in range(_NBUF):
        wait_store(b)


@jax.jit
def kernel(inputs, weight):
    flat_idx = inputs.astype(jnp.int32).T.reshape(_B)
    mesh = plsc.VectorSubcoreMesh(core_axis_name="c", subcore_axis_name="s")
    run = pl.kernel(
        _gather_body,
        out_type=jax.ShapeDtypeStruct((_B, HIDDEN), jnp.float32),
        mesh=mesh,
        scratch_types=(
            [pltpu.VMEM((_PER_W,), jnp.int32)]
            + [pltpu.VMEM((_CHUNK, HIDDEN), jnp.float32) for _ in range(_NBUF)]
            + [pltpu.SemaphoreType.DMA for _ in range(2 * _NBUF)]
        ),
    )
    out = run(flat_idx, weight)
    return out.reshape(HIST, BATCH, HIDDEN).transpose(1, 0, 2)
